# Initial kernel scaffold; baseline (speedup 1.0000x reference)
import jax
import jax.numpy as jnp
from jax.experimental import pallas as pl


def kernel(x, edge_index, edge_attr, W_src, W_dst, W_edge, W_attn, ln_w, ln_b):
    raise NotImplementedError("write your pallas kernel here")


# speedup vs baseline: 9.6999x; 9.6999x over previous
import functools

import jax
import jax.numpy as jnp
from jax import lax
from jax.experimental import pallas as pl
from jax.experimental.pallas import tpu as pltpu

NEG_SLOPE = 0.2


def _proj_body(x_ref, ws_ref, wd_ref, hs_ref, hd_ref):
    x = x_ref[...]
    dn = (((1,), (1,)), ((), ()))
    hs_ref[...] = lax.dot_general(x, ws_ref[...], dn,
                                  preferred_element_type=jnp.float32)
    hd_ref[...] = lax.dot_general(x, wd_ref[...], dn,
                                  preferred_element_type=jnp.float32)


def _alpha_body(hs_ref, hd_ref, ea_ref, we_ref, wa_ref, at_ref, gmax_ref):
    i = pl.program_id(0)
    dn = (((1,), (1,)), ((), ()))
    a = hs_ref[...] + hd_ref[...] + lax.dot_general(
        ea_ref[...], we_ref[...], dn, preferred_element_type=jnp.float32)
    a = jnp.maximum(a, NEG_SLOPE * a)
    at = lax.dot_general(wa_ref[...], a, dn,
                         preferred_element_type=jnp.float32)
    at_ref[...] = at
    m = jnp.max(at)

    @pl.when(i == 0)
    def _():
        gmax_ref[0, 0] = m

    @pl.when(i > 0)
    def _():
        gmax_ref[0, 0] = jnp.maximum(gmax_ref[0, 0], m)


def _final_body(acc_ref, dn_ref, hd_ref, lnw_ref, lnb_ref, y_ref):
    acc = acc_ref[...]
    den = dn_ref[...]
    h = den.shape[1]
    dh = acc.shape[1] // h
    parts = [acc[:, k * dh:(k + 1) * dh] / (den[:, k:k + 1] + 1e-9)
             for k in range(h)]
    y = jnp.concatenate(parts, axis=1) + hd_ref[...]
    mu = jnp.mean(y, axis=1, keepdims=True)
    var = jnp.mean((y - mu) ** 2, axis=1, keepdims=True)
    y = (y - mu) * lax.rsqrt(var + 1e-5)
    y_ref[...] = y * lnw_ref[...] + lnb_ref[...]


def kernel(x, edge_index, edge_attr, W_src, W_dst, W_edge, W_attn, ln_w, ln_b):
    n, din = x.shape
    e = edge_index.shape[1]
    dout = W_src.shape[0]
    de = edge_attr.shape[1]
    h = W_attn.shape[0]
    dh = dout // h
    nb = 1000
    eb = 2560

    h_src, h_dst = pl.pallas_call(
        _proj_body,
        grid=(n // nb,),
        in_specs=[pl.BlockSpec((nb, din), lambda i: (i, 0)),
                  pl.BlockSpec((dout, din), lambda i: (0, 0)),
                  pl.BlockSpec((dout, din), lambda i: (0, 0))],
        out_specs=[pl.BlockSpec((nb, dout), lambda i: (i, 0)),
                   pl.BlockSpec((nb, dout), lambda i: (i, 0))],
        out_shape=[jax.ShapeDtypeStruct((n, dout), jnp.float32)] * 2,
    )(x, W_src, W_dst)

    src = edge_index[0]
    dst = edge_index[1]
    hs_g = jnp.take(h_src, src, axis=0)
    hd_g = jnp.take(h_dst, dst, axis=0)

    alpha_t, gmax = pl.pallas_call(
        _alpha_body,
        grid=(e // eb,),
        in_specs=[pl.BlockSpec((eb, dout), lambda i: (i, 0)),
                  pl.BlockSpec((eb, dout), lambda i: (i, 0)),
                  pl.BlockSpec((eb, de), lambda i: (i, 0)),
                  pl.BlockSpec((dout, de), lambda i: (0, 0)),
                  pl.BlockSpec((h, dout), lambda i: (0, 0))],
        out_specs=[pl.BlockSpec((h, eb), lambda i: (0, i)),
                   pl.BlockSpec((1, 1), lambda i: (0, 0),
                                memory_space=pltpu.SMEM)],
        out_shape=[jax.ShapeDtypeStruct((h, e), jnp.float32),
                   jax.ShapeDtypeStruct((1, 1), jnp.float32)],
    )(hs_g, hd_g, edge_attr, W_edge, W_attn)

    aexp = jnp.exp(alpha_t - gmax[0, 0]).T
    denom = jnp.zeros((n, h), jnp.float32).at[dst].add(aexp)
    accum = jnp.zeros((n, dout), jnp.float32).at[dst].add(
        hs_g * jnp.repeat(aexp, dh, axis=1))

    y = pl.pallas_call(
        _final_body,
        grid=(n // nb,),
        in_specs=[pl.BlockSpec((nb, dout), lambda i: (i, 0)),
                  pl.BlockSpec((nb, h), lambda i: (i, 0)),
                  pl.BlockSpec((nb, dout), lambda i: (i, 0)),
                  pl.BlockSpec((1, dout), lambda i: (0, 0)),
                  pl.BlockSpec((1, dout), lambda i: (0, 0))],
        out_specs=pl.BlockSpec((nb, dout), lambda i: (i, 0)),
        out_shape=jax.ShapeDtypeStruct((n, dout), jnp.float32),
    )(accum, denom, h_dst, ln_w.reshape(1, dout), ln_b.reshape(1, dout))
    return y


# speedup vs baseline: 14.5421x; 1.4992x over previous
import functools

import jax
import jax.numpy as jnp
from jax import lax
from jax.experimental import pallas as pl
from jax.experimental.pallas import tpu as pltpu
from jax.experimental.pallas import tpu_sc as plsc

NEG_SLOPE = 0.2


def _make_sc_gather(n, e, d):
    info = plsc.get_sparse_core_info()
    nc, ns = info.num_cores, info.num_subcores
    nw = nc * ns
    e_per_w = e // nw
    k = 80
    n_chunks = e_per_w // k
    assert e_per_w * nw == e and n_chunks * k == e_per_w
    mesh = plsc.VectorSubcoreMesh(core_axis_name="c", subcore_axis_name="s")

    @functools.partial(
        pl.kernel, mesh=mesh,
        out_type=[jax.ShapeDtypeStruct((e, d), jnp.float32)] * 2,
        scratch_types=[
            pltpu.VMEM((k,), jnp.int32),
            pltpu.VMEM((k,), jnp.int32),
            pltpu.VMEM((k, d), jnp.float32),
            pltpu.VMEM((k, d), jnp.float32),
            pltpu.SemaphoreType.DMA,
        ],
    )
    def gather_k(hs_hbm, hd_hbm, src_hbm, dst_hbm, hs_out, hd_out,
                 sidx, didx, srows, drows, sem):
        wid = lax.axis_index("s") * nc + lax.axis_index("c")
        base = wid * e_per_w

        def body(i, carry):
            off = base + i * k
            pltpu.sync_copy(src_hbm.at[pl.ds(off, k)], sidx)
            pltpu.sync_copy(dst_hbm.at[pl.ds(off, k)], didx)
            cp1 = pltpu.async_copy(hs_hbm.at[sidx], srows, sem)
            cp2 = pltpu.async_copy(hd_hbm.at[didx], drows, sem)
            cp1.wait()
            cp2.wait()
            pltpu.sync_copy(srows, hs_out.at[pl.ds(off, k)])
            pltpu.sync_copy(drows, hd_out.at[pl.ds(off, k)])
            return carry

        lax.fori_loop(0, n_chunks, body, 0)

    return gather_k


def _proj_body(x_ref, ws_ref, wd_ref, hs_ref, hd_ref):
    x = x_ref[...]
    dn = (((1,), (1,)), ((), ()))
    hs_ref[...] = lax.dot_general(x, ws_ref[...], dn,
                                  preferred_element_type=jnp.float32)
    hd_ref[...] = lax.dot_general(x, wd_ref[...], dn,
                                  preferred_element_type=jnp.float32)


def _alpha_body(hs_ref, hd_ref, ea_ref, we_ref, wa_ref, at_ref, gmax_ref):
    i = pl.program_id(0)
    dn = (((1,), (1,)), ((), ()))
    a = hs_ref[...] + hd_ref[...] + lax.dot_general(
        ea_ref[...], we_ref[...], dn, preferred_element_type=jnp.float32)
    a = jnp.maximum(a, NEG_SLOPE * a)
    at = lax.dot_general(wa_ref[...], a, dn,
                         preferred_element_type=jnp.float32)
    at_ref[...] = at
    m = jnp.max(at)

    @pl.when(i == 0)
    def _():
        gmax_ref[0, 0] = m

    @pl.when(i > 0)
    def _():
        gmax_ref[0, 0] = jnp.maximum(gmax_ref[0, 0], m)


def _final_body(acc_ref, dn_ref, hd_ref, lnw_ref, lnb_ref, y_ref):
    acc = acc_ref[...]
    den = dn_ref[...]
    h = den.shape[1]
    dh = acc.shape[1] // h
    parts = [acc[:, k * dh:(k + 1) * dh] / (den[:, k:k + 1] + 1e-9)
             for k in range(h)]
    y = jnp.concatenate(parts, axis=1) + hd_ref[...]
    mu = jnp.mean(y, axis=1, keepdims=True)
    var = jnp.mean((y - mu) ** 2, axis=1, keepdims=True)
    y = (y - mu) * lax.rsqrt(var + 1e-5)
    y_ref[...] = y * lnw_ref[...] + lnb_ref[...]


def kernel(x, edge_index, edge_attr, W_src, W_dst, W_edge, W_attn, ln_w, ln_b):
    n, din = x.shape
    e = edge_index.shape[1]
    dout = W_src.shape[0]
    de = edge_attr.shape[1]
    h = W_attn.shape[0]
    dh = dout // h
    nb = 1000
    eb = 2560

    h_src, h_dst = pl.pallas_call(
        _proj_body,
        grid=(n // nb,),
        in_specs=[pl.BlockSpec((nb, din), lambda i: (i, 0)),
                  pl.BlockSpec((dout, din), lambda i: (0, 0)),
                  pl.BlockSpec((dout, din), lambda i: (0, 0))],
        out_specs=[pl.BlockSpec((nb, dout), lambda i: (i, 0)),
                   pl.BlockSpec((nb, dout), lambda i: (i, 0))],
        out_shape=[jax.ShapeDtypeStruct((n, dout), jnp.float32)] * 2,
    )(x, W_src, W_dst)

    src = edge_index[0]
    dst = edge_index[1]
    hs_g, hd_g = _make_sc_gather(n, e, dout)(h_src, h_dst, src, dst)

    alpha_t, gmax = pl.pallas_call(
        _alpha_body,
        grid=(e // eb,),
        in_specs=[pl.BlockSpec((eb, dout), lambda i: (i, 0)),
                  pl.BlockSpec((eb, dout), lambda i: (i, 0)),
                  pl.BlockSpec((eb, de), lambda i: (i, 0)),
                  pl.BlockSpec((dout, de), lambda i: (0, 0)),
                  pl.BlockSpec((h, dout), lambda i: (0, 0))],
        out_specs=[pl.BlockSpec((h, eb), lambda i: (0, i)),
                   pl.BlockSpec((1, 1), lambda i: (0, 0),
                                memory_space=pltpu.SMEM)],
        out_shape=[jax.ShapeDtypeStruct((h, e), jnp.float32),
                   jax.ShapeDtypeStruct((1, 1), jnp.float32)],
    )(hs_g, hd_g, edge_attr, W_edge, W_attn)

    aexp = jnp.exp(alpha_t - gmax[0, 0]).T
    denom = jnp.zeros((n, h), jnp.float32).at[dst].add(aexp)
    accum = jnp.zeros((n, dout), jnp.float32).at[dst].add(
        hs_g * jnp.repeat(aexp, dh, axis=1))

    y = pl.pallas_call(
        _final_body,
        grid=(n // nb,),
        in_specs=[pl.BlockSpec((nb, dout), lambda i: (i, 0)),
                  pl.BlockSpec((nb, h), lambda i: (i, 0)),
                  pl.BlockSpec((nb, dout), lambda i: (i, 0)),
                  pl.BlockSpec((1, dout), lambda i: (0, 0)),
                  pl.BlockSpec((1, dout), lambda i: (0, 0))],
        out_specs=pl.BlockSpec((nb, dout), lambda i: (i, 0)),
        out_shape=jax.ShapeDtypeStruct((n, dout), jnp.float32),
    )(accum, denom, h_dst, ln_w.reshape(1, dout), ln_b.reshape(1, dout))
    return y


# speedup vs baseline: 21.3222x; 1.4662x over previous
import functools

import jax
import jax.numpy as jnp
from jax import lax
from jax.experimental import pallas as pl
from jax.experimental.pallas import tpu as pltpu
from jax.experimental.pallas import tpu_sc as plsc

NEG_SLOPE = 0.2


def _make_sc_gather(n, e, d):
    info = plsc.get_sparse_core_info()
    nc, ns = info.num_cores, info.num_subcores
    nw = nc * ns
    e_per_w = e // nw
    k = 80
    n_chunks = e_per_w // k
    assert e_per_w * nw == e and n_chunks * k == e_per_w
    mesh = plsc.VectorSubcoreMesh(core_axis_name="c", subcore_axis_name="s")

    @functools.partial(
        pl.kernel, mesh=mesh,
        out_type=[jax.ShapeDtypeStruct((e, d), jnp.float32)] * 2,
        scratch_types=[
            pltpu.VMEM((k,), jnp.int32),
            pltpu.VMEM((k,), jnp.int32),
            pltpu.VMEM((k, d), jnp.float32),
            pltpu.VMEM((k, d), jnp.float32),
            pltpu.SemaphoreType.DMA,
        ],
    )
    def gather_k(hs_hbm, hd_hbm, src_hbm, dst_hbm, hs_out, hd_out,
                 sidx, didx, srows, drows, sem):
        wid = lax.axis_index("s") * nc + lax.axis_index("c")
        base = wid * e_per_w

        def body(i, carry):
            off = base + i * k
            pltpu.sync_copy(src_hbm.at[pl.ds(off, k)], sidx)
            pltpu.sync_copy(dst_hbm.at[pl.ds(off, k)], didx)
            cp1 = pltpu.async_copy(hs_hbm.at[sidx], srows, sem)
            cp2 = pltpu.async_copy(hd_hbm.at[didx], drows, sem)
            cp1.wait()
            cp2.wait()
            pltpu.sync_copy(srows, hs_out.at[pl.ds(off, k)])
            pltpu.sync_copy(drows, hd_out.at[pl.ds(off, k)])
            return carry

        lax.fori_loop(0, n_chunks, body, 0)

    return gather_k


def _make_sc_scatter(n, e, d, nh):
    info = plsc.get_sparse_core_info()
    nc, ns = info.num_cores, info.num_subcores
    nw = nc * ns
    e_per_w = e // nw
    k = 80
    n_chunks = e_per_w // k
    lpr = d // nh
    n2 = ((n + lpr - 1) // lpr + 31) // 32 * 32
    stripe = 8 * ((n + ns - 1) // ns // 8 + 1)
    last_stripe = n - (ns - 1) * stripe
    assert n_chunks * k == e_per_w and 0 < last_stripe <= stripe
    assert stripe % 8 == 0 and last_stripe % 8 == 0
    s2 = 32
    ns2 = n2 // s2
    assert ns2 <= ns and lpr & (lpr - 1) == 0
    mesh = plsc.VectorSubcoreMesh(core_axis_name="c", subcore_axis_name="s")

    @functools.partial(
        pl.kernel, mesh=mesh,
        out_type=[jax.ShapeDtypeStruct((nc, n, d), jnp.float32),
                  jax.ShapeDtypeStruct((nc, n2, d), jnp.float32)],
        scratch_types=[
            pltpu.VMEM_SHARED((n, d), jnp.float32),
            pltpu.VMEM_SHARED((n2, d), jnp.float32),
            pltpu.VMEM_SHARED((k, d), jnp.float32),
            pltpu.VMEM((k,), jnp.int32),
            pltpu.VMEM((k,), jnp.int32),
            pltpu.VMEM((k * d,), jnp.float32),
            pltpu.VMEM((k, d), jnp.float32),
            pltpu.VMEM((k, d), jnp.float32),
            pltpu.VMEM((k * nh + 16,), jnp.float32),
            pltpu.VMEM((32,), jnp.float32),
            pltpu.VMEM((16,), jnp.float32),
        ],
    )
    def scatter_k(hsg_hbm, alpha_hbm, dst_hbm, gmax_hbm, zeros_hbm,
                  out_hbm, out2_hbm,
                  acc, acc2, zscr, didx, didx2, hflat, rows, denrow,
                  aflat, dscr, gv):
        cid = lax.axis_index("c")
        sid = lax.axis_index("s")
        wid = sid * nc + cid
        base = wid * e_per_w

        @pl.when(sid < ns - 1)
        def _():
            pltpu.sync_copy(zeros_hbm.at[pl.ds(sid * stripe, stripe)],
                            acc.at[pl.ds(sid * stripe, stripe)])

        @pl.when(sid == ns - 1)
        def _():
            pltpu.sync_copy(
                zeros_hbm.at[pl.ds((ns - 1) * stripe, last_stripe)],
                acc.at[pl.ds((ns - 1) * stripe, last_stripe)])

        @pl.when(sid < ns2)
        def _():
            pltpu.sync_copy(zeros_hbm.at[pl.ds(0, s2)],
                            acc2.at[pl.ds(sid * s2, s2)])

        @pl.when(sid == 0)
        def _():
            pltpu.sync_copy(zeros_hbm.at[pl.ds(0, k)], zscr)

        pltpu.sync_copy(gmax_hbm, gv)
        plsc.subcore_barrier()
        gvec = gv[...]
        iota16 = lax.iota(jnp.int32, 16)
        mask_nh = iota16 < nh
        zv16 = lax.broadcast_in_dim(jnp.float32(0.0), (16,), ())
        dscr[pl.ds(0, 16)] = zv16
        dscr[pl.ds(16, 16)] = zv16

        def body(i, carry):
            off = base + i * k
            pltpu.sync_copy(dst_hbm.at[pl.ds(off, k)], didx)
            pltpu.sync_copy(hsg_hbm.at[pl.ds(off * d, k * d)], hflat)
            pltpu.sync_copy(alpha_hbm.at[pl.ds(off * nh, k * nh)],
                            aflat.at[pl.ds(0, k * nh)])
            pltpu.sync_copy(zscr, denrow)
            for g in range(k * nh // 16):
                v = aflat[pl.ds(g * 16, 16)]
                aflat[pl.ds(g * 16, 16)] = jnp.exp(v - gvec)
            for g in range(k // 16):
                dv = didx[pl.ds(g * 16, 16)]
                didx2[pl.ds(g * 16, 16)] = lax.shift_right_logical(dv, 5)

            def gbody(g, c2):
                dvec = didx[pl.ds(g * 16, 16)]
                for u in range(16):
                    j = g * 16 + u
                    dj = dvec[u]
                    av = aflat[pl.ds(j * nh, 16)]
                    for hh in range(nh):
                        sv = lax.broadcast_in_dim(av[hh], (16,), ())
                        for cc in range(d // (16 * nh)):
                            colo = hh * (d // nh) + cc * 16
                            v = hflat[pl.ds(j * d + colo, 16)] * sv
                            rows[j, pl.ds(colo, 16)] = v
                    av4 = jnp.where(mask_nh, av, 0.0)
                    s4 = (dj & (nh - 1)) * nh
                    g4 = (dj & (lpr - 1)) >> 2
                    dscr[pl.ds(s4, 16)] = av4
                    w = dscr[pl.ds(0, 16)]
                    dscr[pl.ds(s4, 16)] = zv16
                    denrow[j, pl.ds(g4 * 16, 16)] = w
                return c2

            lax.fori_loop(0, k // 16, gbody, 0)
            pltpu.sync_copy(rows, acc.at[didx], add=True)
            pltpu.sync_copy(denrow, acc2.at[didx2], add=True)
            return carry

        lax.fori_loop(0, n_chunks, body, 0)
        plsc.subcore_barrier()

        @pl.when(sid < ns - 1)
        def _():
            pltpu.sync_copy(acc.at[pl.ds(sid * stripe, stripe)],
                            out_hbm.at[cid, pl.ds(sid * stripe, stripe)])

        @pl.when(sid == ns - 1)
        def _():
            pltpu.sync_copy(
                acc.at[pl.ds((ns - 1) * stripe, last_stripe)],
                out_hbm.at[cid, pl.ds((ns - 1) * stripe, last_stripe)])

        @pl.when(sid < ns2)
        def _():
            pltpu.sync_copy(acc2.at[pl.ds(sid * s2, s2)],
                            out2_hbm.at[cid, pl.ds(sid * s2, s2)])

    return scatter_k


def _proj_body(x_ref, ws_ref, wd_ref, hs_ref, hd_ref):
    x = x_ref[...]
    dn = (((1,), (1,)), ((), ()))
    hs_ref[...] = lax.dot_general(x, ws_ref[...], dn,
                                  preferred_element_type=jnp.float32)
    hd_ref[...] = lax.dot_general(x, wd_ref[...], dn,
                                  preferred_element_type=jnp.float32)


def _alpha_body(hs_ref, hd_ref, ea_ref, we_ref, wa_ref, al_ref, gmax_ref):
    i = pl.program_id(0)
    dn = (((1,), (1,)), ((), ()))
    a = hs_ref[...] + hd_ref[...] + lax.dot_general(
        ea_ref[...], we_ref[...], dn, preferred_element_type=jnp.float32)
    a = jnp.maximum(a, NEG_SLOPE * a)
    al = lax.dot_general(a, wa_ref[...], dn,
                         preferred_element_type=jnp.float32)
    al_ref[...] = al
    m = jnp.max(al)

    @pl.when(i == 0)
    def _():
        gmax_ref[0, 0] = m

    @pl.when(i > 0)
    def _():
        gmax_ref[0, 0] = jnp.maximum(gmax_ref[0, 0], m)


def _final_body(parts_ref, den_ref, hd_ref, lnw_ref, lnb_ref, y_ref):
    nc = parts_ref.shape[0]
    p = parts_ref[0]
    den = den_ref[0]
    for c in range(1, nc):
        p = p + parts_ref[c]
        den = den + den_ref[c]
    d = hd_ref.shape[1]
    nh = den.shape[1]
    dh = d // nh
    parts = [p[:, kk * dh:(kk + 1) * dh] / (den[:, kk:kk + 1] + 1e-9)
             for kk in range(nh)]
    y = jnp.concatenate(parts, axis=1) + hd_ref[...]
    mu = jnp.mean(y, axis=1, keepdims=True)
    var = jnp.mean((y - mu) ** 2, axis=1, keepdims=True)
    y = (y - mu) * lax.rsqrt(var + 1e-5)
    y_ref[...] = y * lnw_ref[...] + lnb_ref[...]


def kernel(x, edge_index, edge_attr, W_src, W_dst, W_edge, W_attn, ln_w, ln_b):
    n, din = x.shape
    e = edge_index.shape[1]
    dout = W_src.shape[0]
    de = edge_attr.shape[1]
    nh = W_attn.shape[0]
    nb = 1000
    eb = 2560

    h_src, h_dst = pl.pallas_call(
        _proj_body,
        grid=(n // nb,),
        in_specs=[pl.BlockSpec((nb, din), lambda i: (i, 0)),
                  pl.BlockSpec((dout, din), lambda i: (0, 0)),
                  pl.BlockSpec((dout, din), lambda i: (0, 0))],
        out_specs=[pl.BlockSpec((nb, dout), lambda i: (i, 0)),
                   pl.BlockSpec((nb, dout), lambda i: (i, 0))],
        out_shape=[jax.ShapeDtypeStruct((n, dout), jnp.float32)] * 2,
    )(x, W_src, W_dst)

    src = edge_index[0]
    dst = edge_index[1]
    hs_g, hd_g = _make_sc_gather(n, e, dout)(h_src, h_dst, src, dst)

    alpha, gmax = pl.pallas_call(
        _alpha_body,
        grid=(e // eb,),
        in_specs=[pl.BlockSpec((eb, dout), lambda i: (i, 0)),
                  pl.BlockSpec((eb, dout), lambda i: (i, 0)),
                  pl.BlockSpec((eb, de), lambda i: (i, 0)),
                  pl.BlockSpec((dout, de), lambda i: (0, 0)),
                  pl.BlockSpec((nh, dout), lambda i: (0, 0))],
        out_specs=[pl.BlockSpec((eb, nh), lambda i: (i, 0)),
                   pl.BlockSpec((1, 1), lambda i: (0, 0),
                                memory_space=pltpu.SMEM)],
        out_shape=[jax.ShapeDtypeStruct((e, nh), jnp.float32),
                   jax.ShapeDtypeStruct((1, 1), jnp.float32)],
    )(hs_g, hd_g, edge_attr, W_edge, W_attn)

    gv16 = jnp.full((16,), gmax[0, 0], jnp.float32)
    zeros = jnp.zeros((n, dout), jnp.float32)
    parts, parts2 = _make_sc_scatter(n, e, dout, nh)(
        hs_g.reshape(-1), alpha.reshape(-1), dst, gv16, zeros)
    den2 = parts2.reshape(2, -1)[:, :n * nh].reshape(2, n, nh)

    y = pl.pallas_call(
        _final_body,
        grid=(n // nb,),
        in_specs=[pl.BlockSpec((2, nb, dout), lambda i: (0, i, 0)),
                  pl.BlockSpec((2, nb, nh), lambda i: (0, i, 0)),
                  pl.BlockSpec((nb, dout), lambda i: (i, 0)),
                  pl.BlockSpec((1, dout), lambda i: (0, 0)),
                  pl.BlockSpec((1, dout), lambda i: (0, 0))],
        out_specs=pl.BlockSpec((nb, dout), lambda i: (i, 0)),
        out_shape=jax.ShapeDtypeStruct((n, dout), jnp.float32),
    )(parts, den2, h_dst, ln_w.reshape(1, dout), ln_b.reshape(1, dout))
    return y


# speedup vs baseline: 21.3254x; 1.0002x over previous
import functools

import jax
import jax.numpy as jnp
from jax import lax
from jax.experimental import pallas as pl
from jax.experimental.pallas import tpu as pltpu
from jax.experimental.pallas import tpu_sc as plsc

NEG_SLOPE = 0.2


def _make_sc_gather(n, e, d):
    info = plsc.get_sparse_core_info()
    nc, ns = info.num_cores, info.num_subcores
    nw = nc * ns
    e_per_w = e // nw
    k = 80
    n_chunks = e_per_w // k
    assert e_per_w * nw == e and n_chunks * k == e_per_w
    mesh = plsc.VectorSubcoreMesh(core_axis_name="c", subcore_axis_name="s")

    @functools.partial(
        pl.kernel, mesh=mesh,
        out_type=[jax.ShapeDtypeStruct((e, d), jnp.float32)] * 2,
        scratch_types=[
            pltpu.VMEM((k,), jnp.int32),
            pltpu.VMEM((k,), jnp.int32),
            pltpu.VMEM((k, d), jnp.float32),
            pltpu.VMEM((k, d), jnp.float32),
            pltpu.SemaphoreType.DMA,
        ],
    )
    def gather_k(hs_hbm, hd_hbm, src_hbm, dst_hbm, hs_out, hd_out,
                 sidx, didx, srows, drows, sem):
        wid = lax.axis_index("s") * nc + lax.axis_index("c")
        base = wid * e_per_w

        def body(i, carry):
            off = base + i * k
            pltpu.sync_copy(src_hbm.at[pl.ds(off, k)], sidx)
            pltpu.sync_copy(dst_hbm.at[pl.ds(off, k)], didx)
            cp1 = pltpu.async_copy(hs_hbm.at[sidx], srows, sem)
            cp2 = pltpu.async_copy(hd_hbm.at[didx], drows, sem)
            cp1.wait()
            cp2.wait()
            pltpu.sync_copy(srows, hs_out.at[pl.ds(off, k)])
            pltpu.sync_copy(drows, hd_out.at[pl.ds(off, k)])
            return carry

        lax.fori_loop(0, n_chunks, body, 0)

    return gather_k


def _make_sc_scatter(n, e, d, nh):
    info = plsc.get_sparse_core_info()
    nc, ns = info.num_cores, info.num_subcores
    nw = nc * ns
    e_per_w = e // nw
    k = 80
    n_chunks = e_per_w // k
    lpr = d // nh
    n2 = ((n + lpr - 1) // lpr + 31) // 32 * 32
    stripe = 8 * ((n + ns - 1) // ns // 8 + 1)
    last_stripe = n - (ns - 1) * stripe
    assert n_chunks * k == e_per_w and 0 < last_stripe <= stripe
    assert stripe % 8 == 0 and last_stripe % 8 == 0
    s2 = 32
    ns2 = n2 // s2
    assert ns2 <= ns and lpr & (lpr - 1) == 0
    mesh = plsc.VectorSubcoreMesh(core_axis_name="c", subcore_axis_name="s")

    @functools.partial(
        pl.kernel, mesh=mesh,
        out_type=[jax.ShapeDtypeStruct((nc, n, d), jnp.float32),
                  jax.ShapeDtypeStruct((nc, n2, d), jnp.float32)],
        scratch_types=[
            pltpu.VMEM_SHARED((n, d), jnp.float32),
            pltpu.VMEM_SHARED((n2, d), jnp.float32),
            pltpu.VMEM_SHARED((k, d), jnp.float32),
            pltpu.VMEM((k,), jnp.int32),
            pltpu.VMEM((k,), jnp.int32),
            pltpu.VMEM((k * d,), jnp.float32),
            pltpu.VMEM((k, d), jnp.float32),
            pltpu.VMEM((k, d), jnp.float32),
            pltpu.VMEM((k * nh + 16,), jnp.float32),
            pltpu.VMEM((32,), jnp.float32),
            pltpu.VMEM((16,), jnp.float32),
        ],
    )
    def scatter_k(hsg_hbm, alpha_hbm, dst_hbm, gmax_hbm, zeros_hbm,
                  out_hbm, out2_hbm,
                  acc, acc2, zscr, didx, didx2, hflat, rows, denrow,
                  aflat, dscr, gv):
        cid = lax.axis_index("c")
        sid = lax.axis_index("s")
        wid = sid * nc + cid
        base = wid * e_per_w

        @pl.when(sid < ns - 1)
        def _():
            pltpu.sync_copy(zeros_hbm.at[pl.ds(sid * stripe, stripe)],
                            acc.at[pl.ds(sid * stripe, stripe)])

        @pl.when(sid == ns - 1)
        def _():
            pltpu.sync_copy(
                zeros_hbm.at[pl.ds((ns - 1) * stripe, last_stripe)],
                acc.at[pl.ds((ns - 1) * stripe, last_stripe)])

        @pl.when(sid < ns2)
        def _():
            pltpu.sync_copy(zeros_hbm.at[pl.ds(0, s2)],
                            acc2.at[pl.ds(sid * s2, s2)])

        @pl.when(sid == 0)
        def _():
            pltpu.sync_copy(zeros_hbm.at[pl.ds(0, k)], zscr)

        pltpu.sync_copy(gmax_hbm, gv)
        plsc.subcore_barrier()
        gvec = gv[...]
        iota16 = lax.iota(jnp.int32, 16)
        mask_nh = iota16 < nh
        zv16 = lax.broadcast_in_dim(jnp.float32(0.0), (16,), ())
        dscr[pl.ds(0, 16)] = zv16
        dscr[pl.ds(16, 16)] = zv16

        def body(i, carry):
            off = base + i * k
            pltpu.sync_copy(dst_hbm.at[pl.ds(off, k)], didx)
            pltpu.sync_copy(hsg_hbm.at[pl.ds(off * d, k * d)], hflat)
            pltpu.sync_copy(alpha_hbm.at[pl.ds(off * nh, k * nh)],
                            aflat.at[pl.ds(0, k * nh)])
            pltpu.sync_copy(zscr, denrow)
            for g in range(k * nh // 16):
                v = aflat[pl.ds(g * 16, 16)]
                aflat[pl.ds(g * 16, 16)] = jnp.exp(v - gvec)
            for g in range(k // 16):
                dv = didx[pl.ds(g * 16, 16)]
                didx2[pl.ds(g * 16, 16)] = lax.shift_right_logical(dv, 5)

            def gbody(g, c2):
                dvec = didx[pl.ds(g * 16, 16)]
                for u in range(16):
                    j = g * 16 + u
                    dj = dvec[u]
                    av = aflat[pl.ds(j * nh, 16)]
                    for hh in range(nh):
                        sv = lax.broadcast_in_dim(av[hh], (16,), ())
                        for cc in range(d // (16 * nh)):
                            colo = hh * (d // nh) + cc * 16
                            rows[j, pl.ds(colo, 16)] = (
                                hflat[pl.ds(j * d + colo, 16)] * sv)
                    av4 = jnp.where(mask_nh, av, 0.0)
                    s4 = (dj & (nh - 1)) * nh
                    g4 = (dj & (lpr - 1)) >> 2
                    dscr[pl.ds(s4, 16)] = av4
                    w = dscr[pl.ds(0, 16)]
                    dscr[pl.ds(s4, 16)] = zv16
                    denrow[j, pl.ds(g4 * 16, 16)] = w
                return c2

            lax.fori_loop(0, k // 16, gbody, 0)
            pltpu.sync_copy(rows, acc.at[didx], add=True)
            pltpu.sync_copy(denrow, acc2.at[didx2], add=True)
            return carry

        lax.fori_loop(0, n_chunks, body, 0)
        plsc.subcore_barrier()

        @pl.when(sid < ns - 1)
        def _():
            pltpu.sync_copy(acc.at[pl.ds(sid * stripe, stripe)],
                            out_hbm.at[cid, pl.ds(sid * stripe, stripe)])

        @pl.when(sid == ns - 1)
        def _():
            pltpu.sync_copy(
                acc.at[pl.ds((ns - 1) * stripe, last_stripe)],
                out_hbm.at[cid, pl.ds((ns - 1) * stripe, last_stripe)])

        @pl.when(sid < ns2)
        def _():
            pltpu.sync_copy(acc2.at[pl.ds(sid * s2, s2)],
                            out2_hbm.at[cid, pl.ds(sid * s2, s2)])

    return scatter_k


def _proj_body(x_ref, ws_ref, wd_ref, hs_ref, hd_ref):
    x = x_ref[...]
    dn = (((1,), (1,)), ((), ()))
    hs_ref[...] = lax.dot_general(x, ws_ref[...], dn,
                                  preferred_element_type=jnp.float32)
    hd_ref[...] = lax.dot_general(x, wd_ref[...], dn,
                                  preferred_element_type=jnp.float32)


def _alpha_body(hs_ref, hd_ref, ea_ref, we_ref, wa_ref, al_ref, gmax_ref):
    i = pl.program_id(0)
    dn = (((1,), (1,)), ((), ()))
    a = hs_ref[...] + hd_ref[...] + lax.dot_general(
        ea_ref[...], we_ref[...], dn, preferred_element_type=jnp.float32)
    a = jnp.maximum(a, NEG_SLOPE * a)
    al = lax.dot_general(a, wa_ref[...], dn,
                         preferred_element_type=jnp.float32)
    al_ref[...] = al
    m = jnp.max(al)

    @pl.when(i == 0)
    def _():
        gmax_ref[0, 0] = m

    @pl.when(i > 0)
    def _():
        gmax_ref[0, 0] = jnp.maximum(gmax_ref[0, 0], m)


def _final_body(parts_ref, den_ref, hd_ref, lnw_ref, lnb_ref, y_ref):
    nc = parts_ref.shape[0]
    p = parts_ref[0]
    den = den_ref[0]
    for c in range(1, nc):
        p = p + parts_ref[c]
        den = den + den_ref[c]
    d = hd_ref.shape[1]
    nh = den.shape[1]
    dh = d // nh
    parts = [p[:, kk * dh:(kk + 1) * dh] / (den[:, kk:kk + 1] + 1e-9)
             for kk in range(nh)]
    y = jnp.concatenate(parts, axis=1) + hd_ref[...]
    mu = jnp.mean(y, axis=1, keepdims=True)
    var = jnp.mean((y - mu) ** 2, axis=1, keepdims=True)
    y = (y - mu) * lax.rsqrt(var + 1e-5)
    y_ref[...] = y * lnw_ref[...] + lnb_ref[...]


def kernel(x, edge_index, edge_attr, W_src, W_dst, W_edge, W_attn, ln_w, ln_b):
    n, din = x.shape
    e = edge_index.shape[1]
    dout = W_src.shape[0]
    de = edge_attr.shape[1]
    nh = W_attn.shape[0]
    nb = 1000
    eb = 2560

    h_src, h_dst = pl.pallas_call(
        _proj_body,
        grid=(n // nb,),
        in_specs=[pl.BlockSpec((nb, din), lambda i: (i, 0)),
                  pl.BlockSpec((dout, din), lambda i: (0, 0)),
                  pl.BlockSpec((dout, din), lambda i: (0, 0))],
        out_specs=[pl.BlockSpec((nb, dout), lambda i: (i, 0)),
                   pl.BlockSpec((nb, dout), lambda i: (i, 0))],
        out_shape=[jax.ShapeDtypeStruct((n, dout), jnp.float32)] * 2,
    )(x, W_src, W_dst)

    src = edge_index[0]
    dst = edge_index[1]
    hs_g, hd_g = _make_sc_gather(n, e, dout)(h_src, h_dst, src, dst)

    alpha, gmax = pl.pallas_call(
        _alpha_body,
        grid=(e // eb,),
        in_specs=[pl.BlockSpec((eb, dout), lambda i: (i, 0)),
                  pl.BlockSpec((eb, dout), lambda i: (i, 0)),
                  pl.BlockSpec((eb, de), lambda i: (i, 0)),
                  pl.BlockSpec((dout, de), lambda i: (0, 0)),
                  pl.BlockSpec((nh, dout), lambda i: (0, 0))],
        out_specs=[pl.BlockSpec((eb, nh), lambda i: (i, 0)),
                   pl.BlockSpec((1, 1), lambda i: (0, 0),
                                memory_space=pltpu.SMEM)],
        out_shape=[jax.ShapeDtypeStruct((e, nh), jnp.float32),
                   jax.ShapeDtypeStruct((1, 1), jnp.float32)],
    )(hs_g, hd_g, edge_attr, W_edge, W_attn)

    gv16 = jnp.full((16,), gmax[0, 0], jnp.float32)
    zeros = jnp.zeros((n, dout), jnp.float32)
    parts, parts2 = _make_sc_scatter(n, e, dout, nh)(
        hs_g.reshape(-1), alpha.reshape(-1), dst, gv16, zeros)
    den2 = parts2.reshape(2, -1)[:, :n * nh].reshape(2, n, nh)

    y = pl.pallas_call(
        _final_body,
        grid=(n // nb,),
        in_specs=[pl.BlockSpec((2, nb, dout), lambda i: (0, i, 0)),
                  pl.BlockSpec((2, nb, nh), lambda i: (0, i, 0)),
                  pl.BlockSpec((nb, dout), lambda i: (i, 0)),
                  pl.BlockSpec((1, dout), lambda i: (0, 0)),
                  pl.BlockSpec((1, dout), lambda i: (0, 0))],
        out_specs=pl.BlockSpec((nb, dout), lambda i: (i, 0)),
        out_shape=jax.ShapeDtypeStruct((n, dout), jnp.float32),
    )(parts, den2, h_dst, ln_w.reshape(1, dout), ln_b.reshape(1, dout))
    return y
